# Initial kernel scaffold; baseline (speedup 1.0000x reference)
import jax
import jax.numpy as jnp
from jax.experimental import pallas as pl


def kernel(x, W, b):
    raise NotImplementedError("write your pallas kernel here")


# speedup vs baseline: 3.4525x; 3.4525x over previous
import jax
import jax.numpy as jnp
from jax import lax
from jax.experimental import pallas as pl
from jax.experimental.pallas import tpu as pltpu

_N_LAYERS = 20
_TOKENS = 4096
_F = 1024
_APPLY_TILE = 512
_N_APPLY = _TOKENS // _APPLY_TILE

_NT = (((1,), (1,)), ((), ()))
_NN = (((1,), (0,)), ((), ()))


def _body(x_ref, w_ref, b_ref, out_ref, q_scr, c_scr):
    i = pl.program_id(0)

    @pl.when(i == 0)
    def _init():
        q_scr[...] = w_ref[0].T
        c_scr[...] = jnp.broadcast_to(b_ref[0], (8, _F))

    @pl.when((i > 0) & (i < _N_LAYERS))
    def _chain():
        w = w_ref[0]
        q_scr[...] = lax.dot_general(
            q_scr[...], w, _NT, preferred_element_type=jnp.float32)
        c_scr[...] = lax.dot_general(
            c_scr[...], w, _NT, preferred_element_type=jnp.float32) + b_ref[0]

    @pl.when(i >= _N_LAYERS)
    def _apply():
        out_ref[...] = lax.dot_general(
            x_ref[...], q_scr[...], _NN,
            preferred_element_type=jnp.float32) + c_scr[0:1, :]


def kernel(x, W, b):
    return pl.pallas_call(
        _body,
        grid=(_N_LAYERS + _N_APPLY,),
        in_specs=[
            pl.BlockSpec((_APPLY_TILE, _F),
                         lambda i: (jnp.maximum(i - _N_LAYERS, 0), 0)),
            pl.BlockSpec((1, _F, _F),
                         lambda i: (jnp.minimum(i, _N_LAYERS - 1), 0, 0)),
            pl.BlockSpec((1, 1, _F),
                         lambda i: (jnp.minimum(i, _N_LAYERS - 1), 0, 0)),
        ],
        out_specs=pl.BlockSpec((_APPLY_TILE, _F),
                               lambda i: (jnp.maximum(i - _N_LAYERS, 0), 0)),
        out_shape=jax.ShapeDtypeStruct((_TOKENS, _F), jnp.float32),
        scratch_shapes=[
            pltpu.VMEM((_F, _F), jnp.float32),
            pltpu.VMEM((8, _F), jnp.float32),
        ],
    )(x, W, b.reshape(_N_LAYERS, 1, _F))
